# baseline (speedup 1.0000x reference)
import functools

import jax
import jax.numpy as jnp
from jax import lax
from jax.experimental import pallas as pl
from jax.experimental.pallas import tpu as pltpu
from jax.experimental.pallas import tpu_sc as plsc

BLK = 256


def _gate_route_body(nexp, nblocks, x_ref, gw_ref, gb_ref,
                     topw_ref, eid_ref, rank_ref, counts_ref, run_ref):
    pid = pl.program_id(0)

    @pl.when(pid == 0)
    def _():
        run_ref[...] = jnp.zeros_like(run_ref)

    xx = x_ref[...]
    bt = xx.shape[0]
    logits = jnp.dot(xx, gw_ref[...], preferred_element_type=jnp.float32)
    logits = logits + gb_ref[...]
    lanes = lax.broadcasted_iota(jnp.int32, logits.shape, 1)
    logits = jnp.where(lanes < nexp, logits, -jnp.inf)
    m = jnp.max(logits, axis=1, keepdims=True)
    ex = jnp.exp(logits - m)
    s = jnp.sum(ex, axis=1, keepdims=True)
    w = ex / s
    m1 = jnp.max(w, axis=1, keepdims=True)
    i1 = jnp.min(jnp.where(w == m1, lanes, 128), axis=1, keepdims=True)
    w2 = jnp.where(lanes == i1, -1.0, w)
    m2 = jnp.max(w2, axis=1, keepdims=True)
    i2 = jnp.min(jnp.where(w2 == m2, lanes, 128), axis=1, keepdims=True)
    topw_ref[...] = jnp.concatenate([m1, m2], axis=1)
    eid_ref[...] = jnp.concatenate([i1, i2], axis=1)

    oh1 = (i1 == lanes).astype(jnp.float32)
    oh2 = (i2 == lanes).astype(jnp.float32)
    O = jnp.concatenate([oh1, oh2], axis=0)
    ba = 2 * bt
    ri = lax.broadcasted_iota(jnp.int32, (ba, ba), 0)
    ci = lax.broadcasted_iota(jnp.int32, (ba, ba), 1)
    tri = (ri > ci).astype(jnp.float32)
    R = jnp.dot(tri, O, preferred_element_type=jnp.float32)
    run = run_ref[...]
    rank_all = jnp.sum(O * (R + run), axis=1, keepdims=True)
    rank_ref[...] = jnp.concatenate(
        [rank_all[:bt], rank_all[bt:]], axis=1).astype(jnp.int32)
    csum = jnp.sum(O, axis=0, keepdims=True)
    run_ref[...] = run + csum

    @pl.when(pid == nblocks - 1)
    def _():
        counts_ref[...] = run + csum


def _offsets_body(nexp, cap, counts_ref, eid_ref, rank_ref, dest_ref, bo_ref):
    lanes = lax.broadcasted_iota(jnp.int32, (1, 128), 1)
    c = jnp.where(lanes < nexp, counts_ref[...], 0.0)
    padded = jnp.ceil(c * (1.0 / BLK)) * BLK
    ri = lax.broadcasted_iota(jnp.int32, (128, 128), 0)
    ci = lax.broadcasted_iota(jnp.int32, (128, 128), 1)
    triu = (ri < ci).astype(jnp.float32)
    off = jnp.dot(padded, triu, preferred_element_type=jnp.float32)

    t = eid_ref.shape[0]
    lanes_t = lax.broadcasted_iota(jnp.int32, (t, 128), 1)
    cols = []
    for k in range(2):
        eidk = eid_ref[:, k:k + 1]
        ohk = (eidk == lanes_t)
        offsel = jnp.sum(jnp.where(ohk, off, 0.0), axis=1, keepdims=True)
        cols.append(rank_ref[:, k:k + 1] + offsel.astype(jnp.int32))
    dest_ref[...] = jnp.concatenate(cols, axis=1)

    start = (lax.broadcasted_iota(jnp.int32, (128, 128), 0) * BLK
             ).astype(jnp.float32)
    elane = lax.broadcasted_iota(jnp.int32, (128, 128), 1)
    hit = (start >= off) & (start < off + padded) & (elane < nexp)
    bo_ref[...] = jnp.sum(jnp.where(hit, elane, 0), axis=1, keepdims=True)


def _gmm_body(owner_ref, xs_ref, ew_ref, eb_ref, out_ref):
    out_ref[...] = (jnp.dot(xs_ref[...].astype(jnp.bfloat16), ew_ref[0],
                            preferred_element_type=jnp.float32)
                    + eb_ref[0])


def kernel(x, gate_w, gate_b, expert_w, expert_b):
    B, S, EMB = x.shape
    NE, _, HID = expert_w.shape
    T = B * S
    A = 2 * T
    CAP = A + NE * BLK
    NB = CAP // BLK
    BT1 = min(512, T)

    x2d = x.reshape(T, EMB)
    gw = jnp.pad(gate_w, ((0, 0), (0, 128 - NE)))
    gb = jnp.pad(gate_b, (0, 128 - NE)).reshape(1, 128)
    ew16 = expert_w.astype(jnp.bfloat16)

    topw, eid, rank, counts = pl.pallas_call(
        functools.partial(_gate_route_body, NE, T // BT1),
        grid=(T // BT1,),
        in_specs=[
            pl.BlockSpec((BT1, EMB), lambda t: (t, 0)),
            pl.BlockSpec((EMB, 128), lambda t: (0, 0)),
            pl.BlockSpec((1, 128), lambda t: (0, 0)),
        ],
        out_specs=[
            pl.BlockSpec((BT1, 2), lambda t: (t, 0)),
            pl.BlockSpec((BT1, 2), lambda t: (t, 0)),
            pl.BlockSpec((BT1, 2), lambda t: (t, 0)),
            pl.BlockSpec((1, 128), lambda t: (0, 0)),
        ],
        out_shape=[
            jax.ShapeDtypeStruct((T, 2), jnp.float32),
            jax.ShapeDtypeStruct((T, 2), jnp.int32),
            jax.ShapeDtypeStruct((T, 2), jnp.int32),
            jax.ShapeDtypeStruct((1, 128), jnp.float32),
        ],
        scratch_shapes=[pltpu.VMEM((1, 128), jnp.float32)],
    )(x2d, gw, gb)

    dest, bo = pl.pallas_call(
        functools.partial(_offsets_body, NE, CAP),
        grid=(1,),
        in_specs=[
            pl.BlockSpec((1, 128), lambda i: (0, 0)),
            pl.BlockSpec((T, 2), lambda i: (0, 0)),
            pl.BlockSpec((T, 2), lambda i: (0, 0)),
        ],
        out_specs=[
            pl.BlockSpec((T, 2), lambda i: (0, 0)),
            pl.BlockSpec((128, 1), lambda i: (0, 0)),
        ],
        out_shape=[
            jax.ShapeDtypeStruct((T, 2), jnp.int32),
            jax.ShapeDtypeStruct((128, 1), jnp.int32),
        ],
    )(counts, eid, rank)

    dest_flat = dest.reshape(A)
    block_owner = bo.reshape(128)[:NB]

    tok = jnp.arange(A, dtype=jnp.int32) // 2
    st = jnp.zeros((CAP,), jnp.int32).at[dest_flat].set(tok)
    xs = x2d[jnp.clip(st, 0, T - 1)]

    out_sorted = pl.pallas_call(
        _gmm_body,
        grid_spec=pltpu.PrefetchScalarGridSpec(
            num_scalar_prefetch=1,
            grid=(NB,),
            in_specs=[
                pl.BlockSpec((BLK, EMB), lambda g, own: (g, 0)),
                pl.BlockSpec((1, EMB, HID), lambda g, own: (own[g], 0, 0)),
                pl.BlockSpec((1, 1, HID), lambda g, own: (own[g], 0, 0)),
            ],
            out_specs=pl.BlockSpec((BLK, HID), lambda g, own: (g, 0)),
        ),
        out_shape=jax.ShapeDtypeStruct((CAP, HID), jnp.float32),
    )(block_owner, xs, ew16, expert_b.reshape(NE, 1, HID))

    out2d = out_sorted[dest_flat]

    return topw.reshape(B, S, 2), out2d.reshape(B, S, 2, HID)
